# baseline (speedup 1.0000x reference)
import jax
import jax.numpy as jnp
from jax import lax
from jax.experimental import pallas as pl
from jax.experimental.pallas import tpu as pltpu
from jax.experimental.pallas import tpu_sc as plsc

_NC = 2
_NS = 16
_L = 16

_B, _F, _H, _WD = 16, 512, 32, 32
_FT = _F // 128
_JT = _WD // 8
_RUNS = _JT * _FT * 8
_NW = _NC * _NS


_S0 = 13


def _body(w_hbm, out_hbm, idx, gbuf, sem):
    c = lax.axis_index("c")
    s = lax.axis_index("s")
    pair = jnp.logical_and(c == 1, s >= _S0)
    idle = jnp.logical_and(c == 0, s >= _S0)
    base = jnp.where(pair, 2 * s, 2 * s + c)

    lane = lax.iota(jnp.int32, _L)

    def build(h, carry):
        n = h * _L + lane
        r = base + (n >> 7)
        nn = n & (_RUNS - 1)
        jt = nn >> 5
        ft = (nn >> 3) & (_FT - 1)
        jin = nn & 7
        j = jt * 8 + jin
        z = jnp.maximum(jnp.abs(_WD // 2 - j) + jnp.abs(_H // 2 - r) - 1, 0)
        idx[pl.ds(h * _L, _L)] = ((z >> 3) << 5) + (ft << 3) + (z & 7)
        return carry

    nrow = jnp.where(pair, 2, 1)
    lax.fori_loop(0, nrow * (_RUNS // _L), build, 0)

    @pl.when(jnp.logical_not(idle))
    def _gather():
        pltpu.async_copy(w_hbm.at[idx.at[pl.ds(0, _RUNS)]],
                         gbuf.at[pl.ds(0, _RUNS)], sem)
        pltpu.make_async_copy(gbuf.at[pl.ds(0, _RUNS)],
                              out_hbm.at[pl.ds(0, _RUNS)], sem).wait()

    @pl.when(pair)
    def _gather2():
        pltpu.async_copy(w_hbm.at[idx.at[pl.ds(_RUNS, _RUNS)]],
                         gbuf.at[pl.ds(_RUNS, _RUNS)], sem)
        pltpu.make_async_copy(gbuf.at[pl.ds(_RUNS, _RUNS)],
                              out_hbm.at[pl.ds(0, _RUNS)], sem).wait()

    @pl.when(jnp.logical_and(jnp.logical_not(idle), jnp.logical_not(pair)))
    def _single_copies():
        copies = [
            pltpu.async_copy(
                gbuf.at[pl.ds(0, _RUNS)],
                out_hbm.at[pl.ds(b * _H * _RUNS + base * _RUNS, _RUNS)], sem)
            for b in range(_B)
        ]
        for cp in copies:
            cp.wait()

    @pl.when(pair)
    def _pair_copies():
        copies = [
            pltpu.async_copy(
                gbuf,
                out_hbm.at[pl.ds(b * _H * _RUNS + base * _RUNS, 2 * _RUNS)], sem)
            for b in range(_B)
        ]
        for cp in copies:
            cp.wait()


def kernel(x, W):
    del x
    mesh = plsc.VectorSubcoreMesh(core_axis_name="c", subcore_axis_name="s")
    f = pl.kernel(
        _body,
        mesh=mesh,
        compiler_params=pltpu.CompilerParams(
            needs_layout_passes=False,
            disable_bounds_checks=True,
            disable_semaphore_checks=True,
            skip_device_barrier=True,
        ),
        out_type=jax.ShapeDtypeStruct((_B * _H * _RUNS, 128), jnp.float32),
        scratch_types=[
            pltpu.VMEM((2 * _RUNS,), jnp.int32),
            pltpu.VMEM((2 * _RUNS, 128), jnp.float32),
            pltpu.SemaphoreType.DMA,
        ],
    )
    wp = W.reshape(_H // 8, 8, _FT, 128).transpose(0, 2, 1, 3)
    out = f(wp.reshape(_H * _FT, 128))
    out = out.reshape(_B, _H, _JT, _FT, 8, 128)
    out = out.transpose(0, 3, 5, 1, 2, 4)
    return out.reshape(_B, _F, _H, _WD)


# speedup vs baseline: 1.2562x; 1.2562x over previous
import jax
import jax.numpy as jnp
from jax import lax
from jax.experimental import pallas as pl
from jax.experimental.pallas import tpu as pltpu
from jax.experimental.pallas import tpu_sc as plsc

_NC = 2
_NS = 16
_L = 16

_B, _F, _H, _WD = 16, 512, 32, 32
_FT = _F // 128
_JT = _WD // 8
_RUNS = _JT * _FT * 8
_NW = _NC * _NS


def _body(w_hbm, out_hbm, idx, gbuf, sem):
    i = lax.axis_index("s") * _NC + lax.axis_index("c")
    di = jnp.abs(_H // 2 - i)

    lane = lax.iota(jnp.int32, _L)

    def build(h, carry):
        n = h * _L + lane
        jt = n >> 5
        ft = (n >> 3) & (_FT - 1)
        jin = n & 7
        j = jt * 8 + jin
        z = jnp.maximum(jnp.abs(_WD // 2 - j) + di - 1, 0)
        idx[pl.ds(h * _L, _L)] = ((z >> 3) << 5) + (ft << 3) + (z & 7)
        return carry

    lax.fori_loop(0, _RUNS // _L, build, 0)

    pltpu.async_copy(w_hbm.at[idx], gbuf, sem).wait()

    def fire(b, carry):
        pltpu.async_copy(gbuf, out_hbm.at[pl.ds(b * _H * _RUNS + i * _RUNS, _RUNS)], sem)
        return carry

    lax.fori_loop(0, _B, fire, 0)

    def drain(b, carry):
        pltpu.make_async_copy(gbuf, out_hbm.at[pl.ds(b * _H * _RUNS + i * _RUNS, _RUNS)], sem).wait()
        return carry

    lax.fori_loop(0, _B, drain, 0)


def kernel(x, W):
    del x
    mesh = plsc.VectorSubcoreMesh(core_axis_name="c", subcore_axis_name="s")
    f = pl.kernel(
        _body,
        mesh=mesh,
        compiler_params=pltpu.CompilerParams(
            needs_layout_passes=False,
            disable_bounds_checks=True,
            disable_semaphore_checks=True,
            skip_device_barrier=True,
        ),
        out_type=jax.ShapeDtypeStruct((_B * _H * _RUNS, 128), jnp.float32),
        scratch_types=[
            pltpu.VMEM((_RUNS,), jnp.int32),
            pltpu.VMEM((_RUNS, 128), jnp.float32),
            pltpu.SemaphoreType.DMA,
        ],
    )
    wp = W.reshape(_H // 8, 8, _FT, 128).transpose(0, 2, 1, 3)
    out = f(wp.reshape(_H * _FT, 128))
    out = out.reshape(_B, _H, _JT, _FT, 8, 128)
    out = out.transpose(0, 3, 5, 1, 2, 4)
    return out.reshape(_B, _F, _H, _WD)
